# baseline (speedup 1.0000x reference)
import functools

import jax
import jax.numpy as jnp
from jax import lax
from jax.experimental import pallas as pl
from jax.experimental.pallas import tpu as pltpu
from jax.experimental.pallas import tpu_sc as plsc

_NC, _NS, _L = 2, 16, 16
_NW = _NC * _NS

_IDX_W = 128


@functools.lru_cache(maxsize=None)
def _build(B, F, V, D):
    rpc = 32
    idxc = rpc * F
    blk = idxc * D
    b_per_w = B // _NW
    chunks = b_per_w // rpc
    w_rows = (b_per_w * F) // _IDX_W
    groups = idxc // _L
    assert (b_per_w * F) % _IDX_W == 0 and w_rows % 8 == 0
    assert idxc % _L == 0

    mesh = plsc.VectorSubcoreMesh(core_axis_name="c", subcore_axis_name="s")

    @functools.partial(
        pl.kernel,
        mesh=mesh,
        compiler_params=pltpu.CompilerParams(
            use_tc_tiling_on_sc=False, needs_layout_passes=False),
        out_type=jax.ShapeDtypeStruct((B * F * D,), jnp.float32),
        scratch_types=[
            pltpu.VMEM((F * V, D), jnp.float32),
            pltpu.VMEM((w_rows, _IDX_W), jnp.int32),
            pltpu.VMEM((13, _IDX_W), jnp.int32),
            pltpu.VMEM((2 * blk,), jnp.float32),
            pltpu.SemaphoreType.DMA,
        ],
    )
    def tok(idx_hbm, off_hbm, tab_hbm, out_hbm, tab_v, idx_v, off_v, rows_v,
            osem):
        wid = lax.axis_index("s") * _NC + lax.axis_index("c")
        pltpu.sync_copy(tab_hbm, tab_v)
        pltpu.sync_copy(off_hbm, off_v)
        pltpu.sync_copy(idx_hbm.at[pl.ds(wid * w_rows, w_rows)], idx_v)
        base_out = wid * (chunks * blk)

        iota = lax.iota(jnp.int32, _L)
        lane32 = iota * D

        def group_body(c, q, _):
            g = c * groups + q
            p = g * _L
            ids = idx_v[p // _IDX_W, pl.ds(lax.rem(p, _IDX_W), _L)]
            pp = lax.rem(p, 13 * _IDX_W)
            offv = off_v[pp // _IDX_W, pl.ds(lax.rem(pp, _IDX_W), _L)]
            rowid = ids + offv
            wa = jnp.full((_L,), (c % 2) * blk + q * (_L * D), jnp.int32) + lane32
            gc = jnp.zeros((_L,), jnp.int32)
            for d in range(D):
                vals = plsc.load_gather(tab_v, [rowid, gc])
                plsc.store_scatter(rows_v, [wa], vals)
                gc = gc + 1
                wa = wa + 1
            return 0

        od = [None, None]
        for c in range(chunks):
            b = c % 2
            if od[b] is not None:
                od[b].wait()
                od[b] = None
            lax.fori_loop(0, groups, functools.partial(group_body, c), 0)
            od[b] = pltpu.async_copy(
                rows_v.at[pl.ds(b * blk, blk)],
                out_hbm.at[pl.ds(base_out + c * blk, blk)],
                osem)
        for b in range(2):
            if od[b] is not None:
                od[b].wait()

    return tok


def kernel(indices, tables):
    B, F = indices.shape
    F2, V, D = tables.shape
    assert F2 == F
    tok = _build(B, F, V, D)
    off = (((jnp.arange(13 * _IDX_W, dtype=jnp.int32) % F) * V)
           .reshape(13, _IDX_W))
    idx2 = indices.astype(jnp.int32).reshape((B * F) // _IDX_W, _IDX_W)
    out = tok(idx2, off, tables.reshape(F * V, D))
    return out.reshape(B, F * D)


# speedup vs baseline: 2.6746x; 2.6746x over previous
import functools

import jax
import jax.numpy as jnp
from jax import lax
from jax.experimental import pallas as pl
from jax.experimental.pallas import tpu as pltpu
from jax.experimental.pallas import tpu_sc as plsc

_NC, _NS, _L = 2, 16, 16
_NW = _NC * _NS

_IDX_W = 128


@functools.lru_cache(maxsize=None)
def _build(B, F, V, D):
    rpc = 32
    idxc = rpc * F
    b_per_w = B // _NW
    chunks = b_per_w // rpc
    assert b_per_w % rpc == 0 and idxc % _L == 0 and D % _L == 0

    mesh = plsc.VectorSubcoreMesh(core_axis_name="c", subcore_axis_name="s")

    @functools.partial(
        pl.kernel,
        mesh=mesh,
        compiler_params=pltpu.CompilerParams(use_tc_tiling_on_sc=False),
        out_type=jax.ShapeDtypeStruct((B, F * D), jnp.float32),
        scratch_types=[
            pltpu.VMEM((2, idxc), jnp.int32),
            pltpu.VMEM((idxc,), jnp.int32),
            pltpu.VMEM((2, idxc, D), jnp.float32),
            pltpu.VMEM((2, rpc, F * D), jnp.float32),
            pltpu.SemaphoreType.DMA,
            pltpu.SemaphoreType.DMA,
        ],
    )
    def tok(idx_hbm, off_hbm, tab_hbm, out_hbm, idx_v, off_v, ga_v, rb_v,
            gsem, osem):
        wid = lax.axis_index("s") * _NC + lax.axis_index("c")
        pltpu.sync_copy(off_hbm, off_v)
        base_idx = wid * (b_per_w * F)
        base_out = wid * b_per_w

        def repack(b):
            def row_body(r, _):
                for f in range(F):
                    for k in range(D // _L):
                        rb_v[b, r, pl.ds(f * D + k * _L, _L)] = (
                            ga_v[b, r * F + f, pl.ds(k * _L, _L)])
                return 0
            lax.fori_loop(0, rpc, row_body, 0)

        def fire_out(c):
            return pltpu.async_copy(
                rb_v.at[c % 2],
                out_hbm.at[pl.ds(base_out + c * rpc, rpc)],
                osem)

        gd = [None, None]
        od = [None, None]
        for c in range(chunks):
            b = c % 2
            if od[b] is not None:
                od[b].wait()
                od[b] = None
            pltpu.sync_copy(
                idx_hbm.at[pl.ds(base_idx + c * idxc, idxc)], idx_v.at[b])
            for k in range(idxc // _L):
                s = pl.ds(k * _L, _L)
                idx_v[b, s] = idx_v[b, s] + off_v[s]
            if c >= 1:
                pb = (c - 1) % 2
                gd[pb].wait()
                gd[pb] = None
                repack(pb)
                od[pb] = fire_out(c - 1)
            gd[b] = pltpu.async_copy(
                tab_hbm.at[idx_v.at[b]], ga_v.at[b], gsem)
        lb = (chunks - 1) % 2
        gd[lb].wait()
        repack(lb)
        od[lb] = fire_out(chunks - 1)
        for b in range(2):
            if od[b] is not None:
                od[b].wait()

    return tok


def kernel(indices, tables):
    B, F = indices.shape
    F2, V, D = tables.shape
    assert F2 == F
    tok = _build(B, F, V, D)
    idxc = 32 * F
    off = (jnp.arange(idxc, dtype=jnp.int32) % F) * V
    idx1 = indices.astype(jnp.int32).reshape(-1)
    return tok(idx1, off, tables.reshape(F * V, D))


# speedup vs baseline: 3.8424x; 1.4367x over previous
import functools

import jax
import jax.numpy as jnp
from jax import lax
from jax.experimental import pallas as pl
from jax.experimental.pallas import tpu as pltpu
from jax.experimental.pallas import tpu_sc as plsc

_NC, _NS, _L = 2, 16, 16
_NW = _NC * _NS

_IDX_W = 128


@functools.lru_cache(maxsize=None)
def _build(B, F, V, D):
    rpc = 64
    idxc = rpc * F
    nir = idxc // _IDX_W
    b_per_w = B // _NW
    chunks = b_per_w // rpc
    w_rows = chunks * nir
    assert idxc % _IDX_W == 0 and b_per_w % rpc == 0 and w_rows % 8 == 0

    mesh = plsc.VectorSubcoreMesh(core_axis_name="c", subcore_axis_name="s")

    @functools.partial(
        pl.kernel,
        mesh=mesh,
        compiler_params=pltpu.CompilerParams(use_tc_tiling_on_sc=False),
        out_type=jax.ShapeDtypeStruct((B * F, D), jnp.float32),
        scratch_types=[
            pltpu.VMEM((w_rows, _IDX_W), jnp.int32),
            pltpu.VMEM((nir, _IDX_W), jnp.int32),
            pltpu.VMEM((2, idxc, D), jnp.float32),
            pltpu.VMEM_SHARED((F * V, D), jnp.float32),
            pltpu.SemaphoreType.DMA,
            pltpu.SemaphoreType.DMA,
        ],
    )
    def tok(idx_hbm, off_hbm, tab_hbm, out_hbm, idx_v, off_v, rows_v, tab_s,
            gsem, osem):
        wid = lax.axis_index("s") * _NC + lax.axis_index("c")
        @pl.when(lax.axis_index("s") == 0)
        def _():
            pltpu.sync_copy(tab_hbm, tab_s)
        pltpu.sync_copy(off_hbm, off_v)
        pltpu.sync_copy(idx_hbm.at[pl.ds(wid * w_rows, w_rows)], idx_v)
        plsc.subcore_barrier()
        base_flat = wid * (chunks * idxc)

        def fire_out(c):
            return pltpu.async_copy(
                rows_v.at[c % 2],
                out_hbm.at[pl.ds(base_flat + c * idxc, idxc)],
                osem)

        gd = [None, None]
        od = [None, None]
        for c in range(chunks):
            b = c % 2
            if od[b] is not None:
                od[b].wait()
                od[b] = None
            for j in range(nir):
                r = c * nir + j
                for k in range(_IDX_W // _L):
                    s = pl.ds(k * _L, _L)
                    idx_v[r, s] = idx_v[r, s] + off_v[j, s]
            if c >= 1:
                pb = (c - 1) % 2
                for cp in gd[pb]:
                    cp.wait()
                gd[pb] = None
                od[pb] = fire_out(c - 1)
            gd[b] = [
                pltpu.async_copy(
                    tab_s.at[idx_v.at[c * nir + j]],
                    rows_v.at[b, pl.ds(j * _IDX_W, _IDX_W)],
                    gsem,
                )
                for j in range(nir)
            ]
        lb = (chunks - 1) % 2
        for cp in gd[lb]:
            cp.wait()
        od[lb] = fire_out(chunks - 1)
        for b in range(2):
            if od[b] is not None:
                od[b].wait()

    return tok


def kernel(indices, tables):
    B, F = indices.shape
    F2, V, D = tables.shape
    assert F2 == F
    tok = _build(B, F, V, D)
    nir = (64 * F) // _IDX_W
    off = (((jnp.arange(nir * _IDX_W, dtype=jnp.int32) % F) * V)
           .reshape(nir, _IDX_W))
    idx2 = indices.astype(jnp.int32).reshape((B * F) // _IDX_W, _IDX_W)
    out = tok(idx2, off, tables.reshape(F * V, D))
    return out.reshape(B, F * D)
